# baseline (speedup 1.0000x reference)
import functools

import jax
import jax.numpy as jnp
from jax import lax
from jax.experimental import pallas as pl
from jax.experimental.pallas import tpu as pltpu
from jax.experimental.pallas import tpu_sc as plsc

H = 512
W = 512
C = 3
B = 32
NSEG = 100
NSEG_PAD = 112
NPIX = H * W
NELEM = NPIX * C
NTILES = 32
PIX_PER_TILE = NPIX // NTILES
ELEMS_PER_TILE = NELEM // NTILES


_MESH = plsc.VectorSubcoreMesh(core_axis_name="c", subcore_axis_name="s")


@functools.partial(
    pl.kernel,
    mesh=_MESH,
    compiler_params=pltpu.CompilerParams(needs_layout_passes=False),
    out_type=(
        jax.ShapeDtypeStruct((NPIX,), jnp.int32),
        jax.ShapeDtypeStruct((NTILES, 16), jnp.float32),
    ),
    scratch_types=[
        pltpu.VMEM((ELEMS_PER_TILE,), jnp.float32),
        pltpu.VMEM((B * NSEG_PAD,), jnp.int32),
        pltpu.VMEM((NSEG_PAD,), jnp.int32),
        pltpu.VMEM((PIX_PER_TILE,), jnp.int32),
        pltpu.VMEM((PIX_PER_TILE,), jnp.int32),
        pltpu.VMEM((16,), jnp.float32),
        pltpu.SemaphoreType.DMA,
        pltpu.SemaphoreType.DMA,
        pltpu.SemaphoreType.DMA,
    ],
)
def _sc_wordmask(img_hbm, seg_hbm, masks_hbm, wm_hbm, part_hbm,
                 img_v, masks_v, lut_v, seg_v, wm_v, acc_v,
                 seg_sem, img_sem, out_sem):
    wid = lax.axis_index("s") * 2 + lax.axis_index("c")

    seg_cp = pltpu.async_copy(
        seg_hbm.at[pl.ds(wid * PIX_PER_TILE, PIX_PER_TILE)], seg_v, seg_sem)
    img_cp = pltpu.async_copy(
        img_hbm.at[pl.ds(wid * ELEMS_PER_TILE, ELEMS_PER_TILE)], img_v,
        img_sem)

    pltpu.sync_copy(masks_hbm, masks_v)
    for g in range(NSEG_PAD // 16):
        word = jnp.zeros((16,), jnp.int32)
        for b in range(B):
            word = word | (masks_v[pl.ds(b * NSEG_PAD + g * 16, 16)] << b)
        lut_v[pl.ds(g * 16, 16)] = word

    seg_cp.wait()

    def gather_body(j, carry):
        base = j * 128
        for u in range(8):
            segv = seg_v[pl.ds(base + u * 16, 16)]
            wm_v[pl.ds(base + u * 16, 16)] = plsc.load_gather(lut_v, [segv])
        return carry

    lax.fori_loop(0, PIX_PER_TILE // 128, gather_body, 0)
    out_cp = pltpu.async_copy(
        wm_v, wm_hbm.at[pl.ds(wid * PIX_PER_TILE, PIX_PER_TILE)], out_sem)

    img_cp.wait()

    def mean_body(i, accs):
        a0, a1, a2, a3 = accs
        base = i * 128
        a0 = a0 + img_v[pl.ds(base, 16)] + img_v[pl.ds(base + 64, 16)]
        a1 = a1 + img_v[pl.ds(base + 16, 16)] + img_v[pl.ds(base + 80, 16)]
        a2 = a2 + img_v[pl.ds(base + 32, 16)] + img_v[pl.ds(base + 96, 16)]
        a3 = a3 + img_v[pl.ds(base + 48, 16)] + img_v[pl.ds(base + 112, 16)]
        return (a0, a1, a2, a3)

    zero = jnp.zeros((16,), jnp.float32)
    a0, a1, a2, a3 = lax.fori_loop(0, ELEMS_PER_TILE // 128, mean_body,
                                   (zero, zero, zero, zero))
    acc_v[...] = (a0 + a1) + (a2 + a3)
    pltpu.sync_copy(acc_v, part_hbm.at[wid])
    out_cp.wait()


def _tc_body(img_ref, wm_ref, part_ref, out_ref):
    i = pl.program_id(0)
    mean = jnp.sum(part_ref[...]) * (1.0 / NELEM)
    img = img_ref[...]
    wm = wm_ref[...]
    for u in range(4):
        bit = jnp.left_shift(jnp.int32(1), 4 * i + u)
        m = (wm & bit) != 0
        out_ref[u] = jnp.where(m[None], mean, img)


_tc_select = pl.pallas_call(
    _tc_body,
    grid=(B // 4,),
    in_specs=[
        pl.BlockSpec((C, H, W), lambda i: (0, 0, 0)),
        pl.BlockSpec((H, W), lambda i: (0, 0)),
        pl.BlockSpec((NTILES, 16), lambda i: (0, 0)),
    ],
    out_specs=pl.BlockSpec((4, C, H, W), lambda i: (i, 0, 0, 0)),
    out_shape=jax.ShapeDtypeStruct((B, C, H, W), jnp.float32),
)


def kernel(image, segments, masks):
    masks_i = jnp.pad(masks.astype(jnp.int32),
                      ((0, 0), (0, NSEG_PAD - NSEG)))
    img_lin = (image.transpose(2, 0, 1)
               .reshape(C, H // 8, 8, W // 128, 128)
               .transpose(0, 1, 3, 2, 4).reshape(-1))
    seg_lin = (segments.reshape(H // 8, 8, W // 128, 128)
               .transpose(0, 2, 1, 3).reshape(-1))
    wm_lin, partials = _sc_wordmask(img_lin, seg_lin, masks_i.reshape(-1))
    wm = (wm_lin.reshape(H // 8, W // 128, 8, 128)
          .transpose(0, 2, 1, 3).reshape(H, W))
    out_p = _tc_select(image.transpose(2, 0, 1), wm, partials)
    return out_p.transpose(0, 2, 3, 1)


# speedup vs baseline: 1.0359x; 1.0359x over previous
import functools

import jax
import jax.numpy as jnp
from jax import lax
from jax.experimental import pallas as pl
from jax.experimental.pallas import tpu as pltpu
from jax.experimental.pallas import tpu_sc as plsc

H = 512
W = 512
C = 3
B = 32
NSEG = 100
NSEG_PAD = 112
NPIX = H * W
NELEM = NPIX * C
NTILES = 32
PIX_PER_TILE = NPIX // NTILES
BPG = 2


_MESH = plsc.VectorSubcoreMesh(core_axis_name="c", subcore_axis_name="s")


@functools.partial(
    pl.kernel,
    mesh=_MESH,
    compiler_params=pltpu.CompilerParams(needs_layout_passes=False),
    out_type=jax.ShapeDtypeStruct((NPIX,), jnp.int32),
    scratch_types=[
        pltpu.VMEM((B * NSEG_PAD,), jnp.int32),
        pltpu.VMEM((NSEG_PAD,), jnp.int32),
        pltpu.VMEM((PIX_PER_TILE,), jnp.int32),
        pltpu.VMEM((PIX_PER_TILE,), jnp.int32),
        pltpu.SemaphoreType.DMA,
    ],
)
def _sc_wordmask(seg_hbm, masks_hbm, wm_hbm,
                 masks_v, lut_v, seg_v, wm_v, seg_sem):
    wid = lax.axis_index("s") * 2 + lax.axis_index("c")

    seg_cp = pltpu.async_copy(
        seg_hbm.at[pl.ds(wid * PIX_PER_TILE, PIX_PER_TILE)], seg_v, seg_sem)

    pltpu.sync_copy(masks_hbm, masks_v)
    for g in range(NSEG_PAD // 16):
        word = jnp.zeros((16,), jnp.int32)
        for b in range(B):
            word = word | (masks_v[pl.ds(b * NSEG_PAD + g * 16, 16)] << b)
        lut_v[pl.ds(g * 16, 16)] = word

    seg_cp.wait()

    def gather_body(j, carry):
        base = j * 128
        for u in range(8):
            segv = seg_v[pl.ds(base + u * 16, 16)]
            wm_v[pl.ds(base + u * 16, 16)] = plsc.load_gather(lut_v, [segv])
        return carry

    lax.fori_loop(0, PIX_PER_TILE // 128, gather_body, 0)
    pltpu.sync_copy(wm_v, wm_hbm.at[pl.ds(wid * PIX_PER_TILE, PIX_PER_TILE)])


def _tc_body(img_ref, wm_ref, out_ref, mean_ref):
    i = pl.program_id(0)
    img = img_ref[...]
    wm = wm_ref[...]

    @pl.when(i == 0)
    def _():
        mean_ref[0] = jnp.sum(img) * (1.0 / NELEM)

    mean = mean_ref[0]
    for u in range(BPG):
        bit = jnp.left_shift(jnp.int32(1), BPG * i + u)
        m = (wm & bit) != 0
        out_ref[u] = jnp.where(m[None], mean, img)


_tc_select = pl.pallas_call(
    _tc_body,
    grid=(B // BPG,),
    in_specs=[
        pl.BlockSpec((C, H, W), lambda i: (0, 0, 0)),
        pl.BlockSpec((H, W), lambda i: (0, 0)),
    ],
    out_specs=pl.BlockSpec((BPG, C, H, W), lambda i: (i, 0, 0, 0)),
    out_shape=jax.ShapeDtypeStruct((B, C, H, W), jnp.float32),
    scratch_shapes=[pltpu.SMEM((1,), jnp.float32)],
)


def kernel(image, segments, masks):
    masks_i = jnp.pad(masks.astype(jnp.int32),
                      ((0, 0), (0, NSEG_PAD - NSEG)))
    seg_lin = (segments.reshape(H // 8, 8, W // 128, 128)
               .transpose(0, 2, 1, 3).reshape(-1))
    wm_lin = _sc_wordmask(seg_lin, masks_i.reshape(-1))
    wm = (wm_lin.reshape(H // 8, W // 128, 8, 128)
          .transpose(0, 2, 1, 3).reshape(H, W))
    out_p = _tc_select(image.transpose(2, 0, 1), wm)
    return out_p.transpose(0, 2, 3, 1)


# speedup vs baseline: 1.0662x; 1.0292x over previous
import functools

import jax
import jax.numpy as jnp
from jax import lax
from jax.experimental import pallas as pl
from jax.experimental.pallas import tpu as pltpu
from jax.experimental.pallas import tpu_sc as plsc

H = 512
W = 512
C = 3
B = 32
NSEG = 100
NSEG_PAD = 112
NPIX = H * W
NELEM = NPIX * C
NTILES = 32
PIX_PER_TILE = NPIX // NTILES
BPG = 2


_MESH = plsc.VectorSubcoreMesh(core_axis_name="c", subcore_axis_name="s")


@functools.partial(
    pl.kernel,
    mesh=_MESH,
    compiler_params=pltpu.CompilerParams(needs_layout_passes=False),
    out_type=jax.ShapeDtypeStruct((NPIX,), jnp.int32),
    scratch_types=[
        pltpu.VMEM((B * NSEG_PAD,), jnp.int32),
        pltpu.VMEM((NSEG_PAD,), jnp.int32),
        pltpu.VMEM((PIX_PER_TILE,), jnp.int32),
        pltpu.VMEM((PIX_PER_TILE,), jnp.int32),
        pltpu.SemaphoreType.DMA,
    ],
)
def _sc_wordmask(seg_hbm, masks_hbm, wm_hbm,
                 masks_v, lut_v, seg_v, wm_v, seg_sem):
    wid = lax.axis_index("s") * 2 + lax.axis_index("c")

    seg_cp = pltpu.async_copy(
        seg_hbm.at[pl.ds(wid * PIX_PER_TILE, PIX_PER_TILE)], seg_v, seg_sem)

    pltpu.sync_copy(masks_hbm, masks_v)
    for g in range(NSEG_PAD // 16):
        word = jnp.zeros((16,), jnp.int32)
        for b in range(B):
            word = word | (masks_v[pl.ds(b * NSEG_PAD + g * 16, 16)] << b)
        lut_v[pl.ds(g * 16, 16)] = word

    seg_cp.wait()

    @plsc.parallel_loop(0, PIX_PER_TILE, step=16, unroll=8)
    def gather_body(j):
        segv = seg_v[pl.ds(j, 16)]
        wm_v[pl.ds(j, 16)] = plsc.load_gather(lut_v, [segv])
    pltpu.sync_copy(wm_v, wm_hbm.at[pl.ds(wid * PIX_PER_TILE, PIX_PER_TILE)])


def _tc_body(img_ref, wm_ref, out_ref, mean_ref):
    i = pl.program_id(0)
    img = img_ref[...]
    wm = wm_ref[...]

    @pl.when(i == 0)
    def _():
        mean_ref[0] = jnp.sum(img) * (1.0 / NELEM)

    mean = mean_ref[0]
    for u in range(BPG):
        bit = jnp.left_shift(jnp.int32(1), BPG * i + u)
        m = (wm & bit) != 0
        out_ref[u] = jnp.where(m[None], mean, img)


_tc_select = pl.pallas_call(
    _tc_body,
    grid=(B // BPG,),
    in_specs=[
        pl.BlockSpec((C, H, W), lambda i: (0, 0, 0)),
        pl.BlockSpec((H, W), lambda i: (0, 0)),
    ],
    out_specs=pl.BlockSpec((BPG, C, H, W), lambda i: (i, 0, 0, 0)),
    out_shape=jax.ShapeDtypeStruct((B, C, H, W), jnp.float32),
    scratch_shapes=[pltpu.SMEM((1,), jnp.float32)],
)


def kernel(image, segments, masks):
    masks_i = jnp.pad(masks.astype(jnp.int32),
                      ((0, 0), (0, NSEG_PAD - NSEG)))
    seg_lin = (segments.reshape(H // 8, 8, W // 128, 128)
               .transpose(0, 2, 1, 3).reshape(-1))
    wm_lin = _sc_wordmask(seg_lin, masks_i.reshape(-1))
    wm = (wm_lin.reshape(H // 8, W // 128, 8, 128)
          .transpose(0, 2, 1, 3).reshape(H, W))
    out_p = _tc_select(image.transpose(2, 0, 1), wm)
    return out_p.transpose(0, 2, 3, 1)
